# baseline (speedup 1.0000x reference)
import functools

import jax
import jax.numpy as jnp
from jax import lax
from jax.experimental import pallas as pl
from jax.experimental.pallas import tpu as pltpu
from jax.experimental.pallas import tpu_sc as plsc

NC = 2
NS = 16
NW = NC * NS
K = 80


def _sc_aggregate(x, src_r, dst_r, n_chunks):
    n, d = x.shape
    rows_per_tile = n // NS

    mesh = plsc.VectorSubcoreMesh(core_axis_name="c", subcore_axis_name="s")
    NB = 3

    @functools.partial(
        pl.kernel,
        out_type=jax.ShapeDtypeStruct((NC, n, d), jnp.float32),
        mesh=mesh,
        scratch_types=[
            pltpu.VMEM_SHARED((n, d), jnp.float32),
            pltpu.VMEM((n_chunks * K,), jnp.int32),
            pltpu.VMEM((NB, K), jnp.int32),
            pltpu.VMEM((NB, K, d), jnp.float32),
        ]
        + [pltpu.SemaphoreType.DMA] * (3 * NB),
    )
    def sc_agg(x_hbm, src_hbm, dst_hbm, out_hbm, agg_s, src_v, dst_v, rows_v,
               *sems):
        gsem = sems[0:NB]
        dsem = sems[NB:2 * NB]
        ssem = sems[2 * NB:3 * NB]
        c = lax.axis_index("c")
        s = lax.axis_index("s")
        wid = s * NC + c

        pltpu.sync_copy(
            x_hbm.at[pl.ds(s * rows_per_tile, rows_per_tile)],
            agg_s.at[pl.ds(s * rows_per_tile, rows_per_tile)],
        )
        pltpu.sync_copy(src_hbm.at[wid], src_v)
        plsc.subcore_barrier()

        def src_chunk(j):
            return src_v.at[pl.ds(pl.multiple_of(j * K, 8), K)]

        def start(j, b):
            pltpu.async_copy(dst_hbm.at[wid, j], dst_v.at[b], dsem[b])
            pltpu.async_copy(x_hbm.at[src_chunk(j)], rows_v.at[b], gsem[b])

        def scatter(j, b):
            pltpu.make_async_copy(dst_hbm.at[wid, 0], dst_v.at[b], dsem[b]).wait()
            pltpu.make_async_copy(
                x_hbm.at[src_chunk(0)], rows_v.at[b], gsem[b]
            ).wait()
            pltpu.async_copy(rows_v.at[b], agg_s.at[dst_v.at[b]], ssem[b],
                             add=True)

        def scatter_wait(b):
            pltpu.make_async_copy(
                rows_v.at[b], agg_s.at[dst_v.at[b]], ssem[b]
            ).wait()

        for b in range(NB):
            start(b, b)

        def body(m, carry):
            j0 = NB * m
            for b in range(NB):
                scatter(j0 + b, b)
            for b in range(NB):
                @pl.when(j0 + b + NB < n_chunks)
                def _(b=b):
                    scatter_wait(b)
                    start(j0 + b + NB, b)
            return carry

        lax.fori_loop(0, n_chunks // NB, body, 0)
        for r in range(n_chunks % NB):
            scatter(NB * (n_chunks // NB) + r, r)
        for b in range(min(NB, n_chunks)):
            scatter_wait(b)
        plsc.subcore_barrier()

        pltpu.sync_copy(
            agg_s.at[pl.ds(s * rows_per_tile, rows_per_tile)],
            out_hbm.at[c, pl.ds(s * rows_per_tile, rows_per_tile)],
        )

    return sc_agg(x, src_r, dst_r)


def _mlp_kernel(a0_ref, a1_ref, x_ref, w1_ref, b1_ref, w2_ref, b2_ref, o_ref):
    h = a0_ref[...] + a1_ref[...] - x_ref[...]
    h = jnp.dot(h, w1_ref[...], preferred_element_type=jnp.float32) + b1_ref[...]
    h = jnp.maximum(h, 0.0)
    o_ref[...] = (
        jnp.dot(h, w2_ref[...], preferred_element_type=jnp.float32) + b2_ref[...]
    )


def kernel(x, edge_index, W1, b1, W2, b2):
    n, d = x.shape
    e = edge_index.shape[1]
    e_per = e // NW
    n_chunks = e_per // K

    src_r = edge_index[0].reshape(NW, e_per)
    dst_r = edge_index[1].reshape(NW, n_chunks, K)

    n_pad = ((n + NS * 8 - 1) // (NS * 8)) * (NS * 8)
    x_pad = jnp.pad(x, ((0, n_pad - n), (0, 0))) if n_pad != n else x

    agg = _sc_aggregate(x_pad, src_r, dst_r, n_chunks)

    rows_blk = 1000
    grid = (n // rows_blk,)
    out = pl.pallas_call(
        _mlp_kernel,
        grid=grid,
        in_specs=[
            pl.BlockSpec((rows_blk, d), lambda i: (i, 0)),
            pl.BlockSpec((rows_blk, d), lambda i: (i, 0)),
            pl.BlockSpec((rows_blk, d), lambda i: (i, 0)),
            pl.BlockSpec((d, d), lambda i: (0, 0)),
            pl.BlockSpec((1, d), lambda i: (0, 0)),
            pl.BlockSpec((d, d), lambda i: (0, 0)),
            pl.BlockSpec((1, d), lambda i: (0, 0)),
        ],
        out_specs=pl.BlockSpec((rows_blk, d), lambda i: (i, 0)),
        out_shape=jax.ShapeDtypeStruct((n, d), jnp.float32),
    )(agg[0], agg[1], x, W1, b1.reshape(1, d), W2, b2.reshape(1, d))
    return out


# speedup vs baseline: 1.1073x; 1.1073x over previous
import functools

import jax
import jax.numpy as jnp
from jax import lax
from jax.experimental import pallas as pl
from jax.experimental.pallas import tpu as pltpu
from jax.experimental.pallas import tpu_sc as plsc

NC = 2
NS = 16
NW = NC * NS
K = 80


def _sc_aggregate(x, src_r, dst_r, n_chunks):
    n, d = x.shape
    rpt = ((n // NS) // 8) * 8
    rem = n - rpt * (NS - 1)

    mesh = plsc.VectorSubcoreMesh(core_axis_name="c", subcore_axis_name="s")

    @functools.partial(
        pl.kernel,
        out_type=jax.ShapeDtypeStruct((NC, n, d), jnp.float32),
        mesh=mesh,
        scratch_types=[
            pltpu.VMEM_SHARED((n, d), jnp.float32),
            pltpu.VMEM((n_chunks * K,), jnp.int32),
            pltpu.VMEM((2, K), jnp.int32),
            pltpu.VMEM((2, K, d), jnp.float32),
            pltpu.SemaphoreType.DMA,
            pltpu.SemaphoreType.DMA,
            pltpu.SemaphoreType.DMA,
            pltpu.SemaphoreType.DMA,
        ],
    )
    def sc_agg(x_hbm, src_hbm, dst_hbm, out_hbm, agg_s, src_v, dst_v, rows_v,
               gsem0, gsem1, dsem0, dsem1):
        c = lax.axis_index("c")
        s = lax.axis_index("s")
        wid = s * NC + c

        @pl.when(s < NS - 1)
        def _():
            pltpu.sync_copy(
                x_hbm.at[pl.ds(s * rpt, rpt)],
                agg_s.at[pl.ds(s * rpt, rpt)],
            )

        @pl.when(s == NS - 1)
        def _():
            pltpu.sync_copy(
                x_hbm.at[pl.ds((NS - 1) * rpt, rem)],
                agg_s.at[pl.ds((NS - 1) * rpt, rem)],
            )

        pltpu.sync_copy(src_hbm.at[wid], src_v)
        plsc.subcore_barrier()

        def src_chunk(j):
            return src_v.at[pl.ds(pl.multiple_of(j * K, 8), K)]

        def start(j, b, gsem, dsem):
            pltpu.async_copy(dst_hbm.at[wid, j], dst_v.at[b], dsem)
            pltpu.async_copy(x_hbm.at[src_chunk(j)], rows_v.at[b], gsem)

        def drain(j, b, gsem, dsem):
            pltpu.make_async_copy(dst_hbm.at[wid, 0], dst_v.at[b], dsem).wait()
            pltpu.make_async_copy(
                x_hbm.at[src_chunk(0)], rows_v.at[b], gsem
            ).wait()
            pltpu.sync_copy(rows_v.at[b], agg_s.at[dst_v.at[b]], add=True)

        start(0, 0, gsem0, dsem0)
        start(1, 1, gsem1, dsem1)

        def body(m, carry):
            j0 = 2 * m
            drain(j0, 0, gsem0, dsem0)

            @pl.when(j0 + 2 < n_chunks)
            def _():
                start(j0 + 2, 0, gsem0, dsem0)

            drain(j0 + 1, 1, gsem1, dsem1)

            @pl.when(j0 + 3 < n_chunks)
            def _():
                start(j0 + 3, 1, gsem1, dsem1)

            return carry

        lax.fori_loop(0, n_chunks // 2, body, 0)
        if n_chunks % 2:
            drain(n_chunks - 1, 0, gsem0, dsem0)
        plsc.subcore_barrier()

        @pl.when(s < NS - 1)
        def _():
            pltpu.sync_copy(
                agg_s.at[pl.ds(s * rpt, rpt)],
                out_hbm.at[c, pl.ds(s * rpt, rpt)],
            )

        @pl.when(s == NS - 1)
        def _():
            pltpu.sync_copy(
                agg_s.at[pl.ds((NS - 1) * rpt, rem)],
                out_hbm.at[c, pl.ds((NS - 1) * rpt, rem)],
            )

    return sc_agg(x, src_r, dst_r)


def _mlp_kernel(a0_ref, a1_ref, x_ref, w1_ref, b1_ref, w2_ref, b2_ref, o_ref):
    h = a0_ref[0] + a1_ref[0] - x_ref[...]
    h = jnp.dot(h, w1_ref[...], preferred_element_type=jnp.float32) + b1_ref[...]
    h = jnp.maximum(h, 0.0)
    o_ref[...] = (
        jnp.dot(h, w2_ref[...], preferred_element_type=jnp.float32) + b2_ref[...]
    )


def kernel(x, edge_index, W1, b1, W2, b2):
    n, d = x.shape
    e = edge_index.shape[1]
    e_per = e // NW
    n_chunks = e_per // K

    src_r = edge_index[0].reshape(NW, e_per)
    dst_r = edge_index[1].reshape(NW, n_chunks, K)

    agg = _sc_aggregate(x, src_r, dst_r, n_chunks)

    rows_blk = 1000
    grid = (n // rows_blk,)
    out = pl.pallas_call(
        _mlp_kernel,
        grid=grid,
        in_specs=[
            pl.BlockSpec((1, rows_blk, d), lambda i: (0, i, 0)),
            pl.BlockSpec((1, rows_blk, d), lambda i: (1, i, 0)),
            pl.BlockSpec((rows_blk, d), lambda i: (i, 0)),
            pl.BlockSpec((d, d), lambda i: (0, 0)),
            pl.BlockSpec((1, d), lambda i: (0, 0)),
            pl.BlockSpec((d, d), lambda i: (0, 0)),
            pl.BlockSpec((1, d), lambda i: (0, 0)),
        ],
        out_specs=pl.BlockSpec((rows_blk, d), lambda i: (i, 0)),
        out_shape=jax.ShapeDtypeStruct((n, d), jnp.float32),
    )(agg, agg, x, W1, b1.reshape(1, d), W2, b2.reshape(1, d))
    return out


# speedup vs baseline: 1.1136x; 1.0057x over previous
import functools

import jax
import jax.numpy as jnp
from jax import lax
from jax.experimental import pallas as pl
from jax.experimental.pallas import tpu as pltpu
from jax.experimental.pallas import tpu_sc as plsc

NC = 2
NS = 16
NW = NC * NS
K = 80


def _sc_aggregate(x, src_r, dst_r, n_chunks):
    n, d = x.shape
    rpt = ((n // NS) // 8) * 8
    rem = n - rpt * (NS - 1)

    mesh = plsc.VectorSubcoreMesh(core_axis_name="c", subcore_axis_name="s")

    @functools.partial(
        pl.kernel,
        out_type=jax.ShapeDtypeStruct((NC, n, d), jnp.float32),
        mesh=mesh,
        scratch_types=[
            pltpu.VMEM_SHARED((n, d), jnp.float32),
            pltpu.VMEM((n_chunks * K,), jnp.int32),
            pltpu.VMEM((2, K), jnp.int32),
            pltpu.VMEM((2, K, d), jnp.float32),
            pltpu.SemaphoreType.DMA,
            pltpu.SemaphoreType.DMA,
            pltpu.SemaphoreType.DMA,
            pltpu.SemaphoreType.DMA,
        ],
    )
    def sc_agg(x_hbm, src_hbm, dst_hbm, out_hbm, agg_s, src_v, dst_v, rows_v,
               gsem0, gsem1, dsem0, dsem1):
        c = lax.axis_index("c")
        s = lax.axis_index("s")
        wid = s * NC + c

        pltpu.sync_copy(src_hbm.at[wid], src_v)

        def src_chunk(j):
            return src_v.at[pl.ds(pl.multiple_of(j * K, 8), K)]

        def start(j, b, gsem, dsem):
            pltpu.async_copy(dst_hbm.at[wid, j], dst_v.at[b], dsem)
            pltpu.async_copy(x_hbm.at[src_chunk(j)], rows_v.at[b], gsem)

        def drain(j, b, gsem, dsem):
            pltpu.make_async_copy(dst_hbm.at[wid, 0], dst_v.at[b], dsem).wait()
            pltpu.make_async_copy(
                x_hbm.at[src_chunk(0)], rows_v.at[b], gsem
            ).wait()
            pltpu.sync_copy(rows_v.at[b], agg_s.at[dst_v.at[b]], add=True)

        start(0, 0, gsem0, dsem0)
        start(1, 1, gsem1, dsem1)

        @pl.when(s < NS - 1)
        def _():
            pltpu.sync_copy(
                x_hbm.at[pl.ds(s * rpt, rpt)],
                agg_s.at[pl.ds(s * rpt, rpt)],
            )

        @pl.when(s == NS - 1)
        def _():
            pltpu.sync_copy(
                x_hbm.at[pl.ds((NS - 1) * rpt, rem)],
                agg_s.at[pl.ds((NS - 1) * rpt, rem)],
            )

        plsc.subcore_barrier()


        def body(m, carry):
            j0 = 2 * m
            drain(j0, 0, gsem0, dsem0)

            @pl.when(j0 + 2 < n_chunks)
            def _():
                start(j0 + 2, 0, gsem0, dsem0)

            drain(j0 + 1, 1, gsem1, dsem1)

            @pl.when(j0 + 3 < n_chunks)
            def _():
                start(j0 + 3, 1, gsem1, dsem1)

            return carry

        lax.fori_loop(0, n_chunks // 2, body, 0)
        if n_chunks % 2:
            drain(n_chunks - 1, 0, gsem0, dsem0)
        plsc.subcore_barrier()

        @pl.when(s < NS - 1)
        def _():
            pltpu.sync_copy(
                agg_s.at[pl.ds(s * rpt, rpt)],
                out_hbm.at[c, pl.ds(s * rpt, rpt)],
            )

        @pl.when(s == NS - 1)
        def _():
            pltpu.sync_copy(
                agg_s.at[pl.ds((NS - 1) * rpt, rem)],
                out_hbm.at[c, pl.ds((NS - 1) * rpt, rem)],
            )

    return sc_agg(x, src_r, dst_r)


def _mlp_kernel(a0_ref, a1_ref, x_ref, w1_ref, b1_ref, w2_ref, b2_ref, o_ref):
    h = a0_ref[0] + a1_ref[0] - x_ref[...]
    h = jnp.dot(h, w1_ref[...], preferred_element_type=jnp.float32) + b1_ref[...]
    h = jnp.maximum(h, 0.0)
    o_ref[...] = (
        jnp.dot(h, w2_ref[...], preferred_element_type=jnp.float32) + b2_ref[...]
    )


def kernel(x, edge_index, W1, b1, W2, b2):
    n, d = x.shape
    e = edge_index.shape[1]
    e_per = e // NW
    n_chunks = e_per // K

    src_r = edge_index[0].reshape(NW, e_per)
    dst_r = edge_index[1].reshape(NW, n_chunks, K)

    agg = _sc_aggregate(x, src_r, dst_r, n_chunks)

    rows_blk = 1000
    grid = (n // rows_blk,)
    out = pl.pallas_call(
        _mlp_kernel,
        grid=grid,
        in_specs=[
            pl.BlockSpec((1, rows_blk, d), lambda i: (0, i, 0)),
            pl.BlockSpec((1, rows_blk, d), lambda i: (1, i, 0)),
            pl.BlockSpec((rows_blk, d), lambda i: (i, 0)),
            pl.BlockSpec((d, d), lambda i: (0, 0)),
            pl.BlockSpec((1, d), lambda i: (0, 0)),
            pl.BlockSpec((d, d), lambda i: (0, 0)),
            pl.BlockSpec((1, d), lambda i: (0, 0)),
        ],
        out_specs=pl.BlockSpec((rows_blk, d), lambda i: (i, 0)),
        out_shape=jax.ShapeDtypeStruct((n, d), jnp.float32),
    )(agg, agg, x, W1, b1.reshape(1, d), W2, b2.reshape(1, d))
    return out
